# Initial kernel scaffold; baseline (speedup 1.0000x reference)
import jax
import jax.numpy as jnp
from jax.experimental import pallas as pl


def kernel(feat_src, feat_dst, edge_index, W_self, b_self, W_neigh, b_neigh):
    raise NotImplementedError("write your pallas kernel here")


# speedup vs baseline: 7.2272x; 7.2272x over previous
import functools

import jax
import jax.numpy as jnp
from jax import lax
from jax.experimental import pallas as pl
from jax.experimental.pallas import tpu as pltpu
from jax.experimental.pallas import tpu_sc as plsc

N_SRC = 10000
N_DST = 10000
E = 320000
D = 128

NC = 2
NS = 16
L = 16
NW = NC * NS
CHUNK = 128
NROWS = E // CHUNK
STEPS = -(-NROWS // NW)
N_PAD = 10240
HR = N_PAD // D
ROWS_PER_TILE = N_PAD // NS
WB = 64
K = 1

_sc_mesh = plsc.VectorSubcoreMesh(
    core_axis_name="c", subcore_axis_name="s", num_cores=NC, num_subcores=NS)


@functools.partial(
    pl.kernel,
    out_type=(
        jax.ShapeDtypeStruct((NC, N_PAD, D), jnp.float32),
        jax.ShapeDtypeStruct((NC, HR, D), jnp.float32),
    ),
    mesh=_sc_mesh,
    compiler_params=pltpu.CompilerParams(
        use_tc_tiling_on_sc=False, needs_layout_passes=False),
    scratch_types=[
        pltpu.VMEM((K, CHUNK), jnp.int32),
        pltpu.VMEM((K, CHUNK), jnp.int32),
        pltpu.VMEM((K, CHUNK, D), jnp.float32),
        pltpu.VMEM((HR, D), jnp.float32),
        pltpu.VMEM((1, HR), jnp.int32),
        pltpu.VMEM((WB, D), jnp.float32),
        pltpu.VMEM_SHARED((N_PAD, D), jnp.float32),
        pltpu.VMEM_SHARED((HR, D), jnp.float32),
        pltpu.SemaphoreType.DMA,
    ],
)
def _sc_aggregate(feat_hbm, src_hbm, dst_hbm, psum_hbm, pdeg_hbm,
                  idx_s, idx_d, rows, hist, idrows, stg_f,
                  acc_sp, deg_sp, gsem):
    cid = lax.axis_index("c")
    sid = lax.axis_index("s")
    wid = sid * NC + cid
    base = sid * ROWS_PER_TILE
    ones16 = jnp.ones((L,), jnp.float32)

    def _fill_f(i, _):
        stg_f[i // (D // L), pl.ds((i % (D // L)) * L, L)] = (
            jnp.zeros((L,), jnp.float32))
        return 0
    lax.fori_loop(0, WB * (D // L), _fill_f, 0)

    def _fill_h(i, _):
        hist[i // (D // L), pl.ds((i % (D // L)) * L, L)] = (
            jnp.zeros((L,), jnp.float32))
        return 0
    lax.fori_loop(0, HR * (D // L), _fill_h, 0)

    for j in range(HR // L):
        idrows[0, pl.ds(j * L, L)] = lax.iota(jnp.int32, L) + (j * L)

    def _zinit(i, _):
        pltpu.sync_copy(stg_f, acc_sp.at[pl.ds(base + i * WB, WB)])
        return 0
    lax.fori_loop(0, ROWS_PER_TILE // WB, _zinit, 0)

    @pl.when(sid == 0)
    def _():
        pltpu.sync_copy(hist, deg_sp)
    plsc.subcore_barrier()

    def _step(k, _):
        r = wid + k * NW

        @pl.when(r < NROWS)
        def _():
            pltpu.sync_copy(src_hbm.at[r], idx_s.at[0])
            pltpu.sync_copy(dst_hbm.at[r], idx_d.at[0])
            pltpu.async_copy(feat_hbm.at[idx_s.at[0]], rows.at[0], gsem).wait()
            pltpu.sync_copy(rows.at[0], acc_sp.at[idx_d.at[0]], add=True)
            for j in range(CHUNK // L):
                idxv = idx_d[0, pl.ds(j * L, L)]
                rowi = lax.shift_right_logical(idxv, 7)
                coli = lax.bitwise_and(idxv, D - 1)
                plsc.addupdate_scatter(hist, [rowi, coli], ones16)
        return 0
    lax.fori_loop(0, STEPS, _step, 0)

    pltpu.sync_copy(hist, deg_sp.at[idrows.at[0]], add=True)
    plsc.subcore_barrier()

    def _flush(i, _):
        off = base + i * WB
        pltpu.sync_copy(acc_sp.at[pl.ds(off, WB)], stg_f)
        pltpu.sync_copy(stg_f, psum_hbm.at[cid, pl.ds(off, WB)])
        return 0
    lax.fori_loop(0, ROWS_PER_TILE // WB, _flush, 0)

    @pl.when(sid == 0)
    def _():
        pltpu.sync_copy(deg_sp, hist)
        pltpu.sync_copy(hist, pdeg_hbm.at[cid])


BLK = 640


def _tc_body(psum_ref, deg_ref, fdst_ref, ws_ref, wn_ref, b_ref, out_ref):
    neigh_sum = psum_ref[0] + psum_ref[1]
    deg = deg_ref[0] + deg_ref[1]
    h_neigh = neigh_sum / jnp.maximum(deg, 1.0)
    self_proj = lax.dot_general(fdst_ref[...], ws_ref[...],
                                (((1,), (1,)), ((), ())),
                                preferred_element_type=jnp.float32)
    neigh_proj = lax.dot_general(h_neigh, wn_ref[...],
                                 (((1,), (1,)), ((), ())),
                                 preferred_element_type=jnp.float32)
    out_ref[...] = self_proj + neigh_proj + b_ref[...]


_tc_combine = pl.pallas_call(
    _tc_body,
    grid=(N_PAD // BLK,),
    in_specs=[
        pl.BlockSpec((NC, BLK, D), lambda i: (0, i, 0)),
        pl.BlockSpec((NC, BLK, 1), lambda i: (0, i, 0)),
        pl.BlockSpec((BLK, D), lambda i: (i, 0)),
        pl.BlockSpec((D, D), lambda i: (0, 0)),
        pl.BlockSpec((D, D), lambda i: (0, 0)),
        pl.BlockSpec((1, D), lambda i: (0, 0)),
    ],
    out_specs=pl.BlockSpec((BLK, D), lambda i: (i, 0)),
    out_shape=jax.ShapeDtypeStruct((N_PAD, D), jnp.float32),
)


def kernel(feat_src, feat_dst, edge_index, W_self, b_self, W_neigh, b_neigh):
    src = edge_index[0].astype(jnp.int32).reshape(NROWS, CHUNK)
    dst = edge_index[1].astype(jnp.int32).reshape(NROWS, CHUNK)
    psum, pdeg = _sc_aggregate(feat_src, src, dst)
    deg_col = pdeg.reshape(NC, N_PAD, 1)
    bias = (b_self + b_neigh).reshape(1, D)
    rst = _tc_combine(psum, deg_col, feat_dst, W_self, W_neigh, bias)
    return rst[:N_DST]
